# baseline (speedup 1.0000x reference)
import functools

import jax
import jax.numpy as jnp
from jax import lax
from jax.experimental import pallas as pl
from jax.experimental.pallas import tpu as pltpu
from jax.experimental.pallas import tpu_sc as plsc

N = 10000
D = 128
E = 320000
NC = 2
NS = 16
NW = NC * NS
R = 10240
CH = 128
EPT = E // NW
NCHUNK = 80
NCPAD = NCHUNK + 2
EPT_PAD = NCPAD * CH
RPS = R // NS
LANES = D // 16
BM = 1024
GRID = R // BM
SBN = 0.9999950000374997

_MESH = dict(core_axis_name="c", subcore_axis_name="s", num_cores=NC,
             num_subcores=NS)


def _worker(c, s):
    return c * NS + s


@functools.partial(
    pl.kernel,
    out_type=jax.ShapeDtypeStruct((NC, R), jnp.float32),
    mesh=plsc.VectorSubcoreMesh(**_MESH),
    scratch_types=[
        pltpu.VMEM((NCPAD, CH), jnp.int32),
        pltpu.VMEM((CH,), jnp.float32),
        pltpu.VMEM((RPS,), jnp.float32),
        pltpu.VMEM_SHARED((R,), jnp.float32),
    ],
)
def _sc_degree(dst_hbm, out_hbm, didx, ones, stripe, acc):
    c = lax.axis_index("c")
    s = lax.axis_index("s")
    wid = _worker(c, s)
    pltpu.sync_copy(dst_hbm.at[wid], didx)

    def fill(i, carry):
        ones[pl.ds(i * 16, 16)] = jnp.full((16,), 1.0, jnp.float32)
        return carry
    lax.fori_loop(0, CH // 16, fill, 0)

    def zero(i, carry):
        stripe[pl.ds(i * 16, 16)] = jnp.zeros((16,), jnp.float32)
        return carry
    lax.fori_loop(0, RPS // 16, zero, 0)
    pltpu.sync_copy(stripe, acc.at[pl.ds(s * RPS, RPS)])
    plsc.subcore_barrier()

    def body(j, carry):
        pltpu.sync_copy(ones, acc.at[didx.at[j]], add=True)
        return carry
    lax.fori_loop(0, NCHUNK, body, 0)
    plsc.subcore_barrier()

    pltpu.sync_copy(acc.at[pl.ds(s * RPS, RPS)], stripe)
    pltpu.sync_copy(stripe, out_hbm.at[c, pl.ds(s * RPS, RPS)])


@functools.partial(
    pl.kernel,
    out_type=jax.ShapeDtypeStruct((NC, R, D), jnp.float32),
    mesh=plsc.VectorSubcoreMesh(**_MESH),
    scratch_types=[
        pltpu.VMEM((NCPAD, CH), jnp.int32),
        pltpu.VMEM((2, CH), jnp.int32),
        pltpu.VMEM((2, CH, D), jnp.float32),
        pltpu.VMEM_SHARED((R, D), jnp.float32),
        pltpu.SemaphoreType.DMA,
        pltpu.SemaphoreType.DMA,
        pltpu.SemaphoreType.DMA,
        pltpu.SemaphoreType.DMA,
    ],
)
def _sc_aggregate(y_hbm, src_hbm, dst_hbm, out_hbm, sidx, didx, rows,
                  acc, rsem0, rsem1, dsem0, dsem1):
    rsems = (rsem0, rsem1)
    dsems = (dsem0, dsem1)
    c = lax.axis_index("c")
    s = lax.axis_index("s")
    wid = _worker(c, s)
    pltpu.sync_copy(src_hbm.at[wid], sidx)

    def zrow(r, carry):
        for j in range(LANES):
            rows[0, r, pl.ds(j * 16, 16)] = jnp.zeros((16,), jnp.float32)
        return carry
    lax.fori_loop(0, CH, zrow, 0)

    def zacc(k, carry):
        pltpu.sync_copy(rows.at[0],
                        acc.at[pl.ds((s * (RPS // CH) + k) * CH, CH)])
        return carry
    lax.fori_loop(0, RPS // CH, zacc, 0)
    plsc.subcore_barrier()

    def _issue(j, p):
        pltpu.async_copy(y_hbm.at[sidx.at[j]], rows.at[p], rsems[p])
        pltpu.async_copy(dst_hbm.at[wid, j], didx.at[p], dsems[p])

    def _wait(j, p):
        pltpu.make_async_copy(y_hbm.at[sidx.at[j]], rows.at[p],
                              rsems[p]).wait()
        pltpu.make_async_copy(dst_hbm.at[wid, j], didx.at[p],
                              dsems[p]).wait()

    _issue(0, 0)
    _issue(1, 1)

    def pair(g, carry):
        for p in range(2):
            j = g * 2 + p
            _wait(j, p)
            pltpu.sync_copy(rows.at[p], acc.at[didx.at[p]], add=True)
            _issue(j + 2, p)
        return carry
    lax.fori_loop(0, NCHUNK // 2, pair, 0)
    for p in range(2):
        _wait(NCHUNK + p, p)
    plsc.subcore_barrier()

    def wout(k, carry):
        off = (s * (RPS // CH) + k) * CH
        pltpu.sync_copy(acc.at[pl.ds(off, CH)], rows.at[0])
        pltpu.sync_copy(rows.at[0], out_hbm.at[c, pl.ds(off, CH)])
        return carry
    lax.fori_loop(0, RPS // CH, wout, 0)


def _row_spec():
    return pl.BlockSpec((BM, D), lambda i: (i, 0))


def _vec_spec():
    return pl.BlockSpec((BM,), lambda i: (i,))


def _full_spec(shape):
    return pl.BlockSpec(shape, lambda i: tuple(0 for _ in shape))


def _tc_matmul_body(x_ref, w_ref, u_ref):
    u_ref[...] = jnp.dot(x_ref[...], w_ref[...],
                         preferred_element_type=jnp.float32)


def _tc_matmul(xp, w1):
    return pl.pallas_call(
        _tc_matmul_body,
        grid=(GRID,),
        in_specs=[_row_spec(), _full_spec((D, D))],
        out_specs=_row_spec(),
        out_shape=jax.ShapeDtypeStruct((R, D), jnp.float32),
    )(xp, w1)


def _tc_scale_body(u_ref, d0_ref, d1_ref, dinv_ref, y_ref):
    dv = lax.rsqrt(d0_ref[...] + d1_ref[...] + 1.0)
    dinv_ref[...] = dv
    y_ref[...] = dv[:, None] * u_ref[...]


def _tc_scale(u1, deg0, deg1):
    return pl.pallas_call(
        _tc_scale_body,
        grid=(GRID,),
        in_specs=[_row_spec(), _vec_spec(), _vec_spec()],
        out_specs=[_vec_spec(), _row_spec()],
        out_shape=[jax.ShapeDtypeStruct((R,), jnp.float32),
                   jax.ShapeDtypeStruct((R, D), jnp.float32)],
    )(u1, deg0, deg1)


def _tc_mid_body(a0_ref, a1_ref, y_ref, dv_ref, b_ref, g_ref, bt_ref, w_ref,
                 out_ref):
    dv = dv_ref[...]
    z = dv[:, None] * (a0_ref[...] + a1_ref[...] + y_ref[...]) + b_ref[...]
    h = jnp.maximum(z * SBN * g_ref[...] + bt_ref[...], 0.0)
    out_ref[...] = dv[:, None] * jnp.dot(h, w_ref[...],
                                         preferred_element_type=jnp.float32)


def _tc_mid(a0, a1, y, dinv, b, g, bt, w_next):
    return pl.pallas_call(
        _tc_mid_body,
        grid=(GRID,),
        in_specs=[_row_spec(), _row_spec(), _row_spec(), _vec_spec(),
                  _full_spec((D,)), _full_spec((D,)), _full_spec((D,)),
                  _full_spec((D, D))],
        out_specs=_row_spec(),
        out_shape=jax.ShapeDtypeStruct((R, D), jnp.float32),
    )(a0, a1, y, dinv, b, g, bt, w_next)


def _tc_final_body(a0_ref, a1_ref, y_ref, dv_ref, b_ref, out_ref):
    dv = dv_ref[...]
    z = dv[:, None] * (a0_ref[...] + a1_ref[...] + y_ref[...]) + b_ref[...]
    z = jnp.clip(z, -4.0, 4.0)
    out_ref[...] = 1.0 / (1.0 + jnp.exp(-z))


def _tc_final(a0, a1, y, dinv, b):
    return pl.pallas_call(
        _tc_final_body,
        grid=(GRID,),
        in_specs=[_row_spec(), _row_spec(), _row_spec(), _vec_spec(),
                  _full_spec((D,))],
        out_specs=_row_spec(),
        out_shape=jax.ShapeDtypeStruct((R, D), jnp.float32),
    )(a0, a1, y, dinv, b)


def kernel(x, edge_index, W1, b1, W2, b2, W3, b3, g1, bt1, g2, bt2):
    src, dst = edge_index[0], edge_index[1]
    npad = EPT_PAD - EPT
    pad_src = jnp.broadcast_to((jnp.arange(npad, dtype=jnp.int32) * 37) % N,
                               (NW, npad))
    pad_dst = jnp.broadcast_to(N + jnp.arange(npad, dtype=jnp.int32) % (R - N),
                               (NW, npad))
    src_p = jnp.concatenate([src.reshape(NW, EPT), pad_src], axis=1)
    dst_p = jnp.concatenate([dst.reshape(NW, EPT), pad_dst], axis=1)
    src_t = src_p.reshape(NW, NCPAD, CH)
    dst_t = dst_p.reshape(NW, NCPAD, CH)
    xp = jnp.pad(x, ((0, R - N), (0, 0)))

    deg = _sc_degree(dst_t)
    u1 = _tc_matmul(xp, W1)
    dinv, y1 = _tc_scale(u1, deg[0], deg[1])
    a1 = _sc_aggregate(y1, src_t, dst_t)
    y2 = _tc_mid(a1[0], a1[1], y1, dinv, b1, g1, bt1, W2)
    a2 = _sc_aggregate(y2, src_t, dst_t)
    y3 = _tc_mid(a2[0], a2[1], y2, dinv, b2, g2, bt2, W3)
    a3 = _sc_aggregate(y3, src_t, dst_t)
    out = _tc_final(a3[0], a3[1], y3, dinv, b3)
    return out[:N]


# speedup vs baseline: 1.0245x; 1.0245x over previous
import functools

import jax
import jax.numpy as jnp
from jax import lax
from jax.experimental import pallas as pl
from jax.experimental.pallas import tpu as pltpu
from jax.experimental.pallas import tpu_sc as plsc

N = 10000
D = 128
E = 320000
NC = 2
NS = 16
NW = NC * NS
R = 10240
CH = 128
EPT = E // NW
NCHUNK = 80
NCPAD = NCHUNK + 2
EPT_PAD = NCPAD * CH
RPS = R // NS
LANES = D // 16
BM = 1024
GRID = R // BM
SBN = 0.9999950000374997

_MESH = dict(core_axis_name="c", subcore_axis_name="s", num_cores=NC,
             num_subcores=NS)


def _worker(c, s):
    return c * NS + s


@functools.partial(
    pl.kernel,
    out_type=jax.ShapeDtypeStruct((NC, R), jnp.float32),
    mesh=plsc.VectorSubcoreMesh(**_MESH),
    scratch_types=[
        pltpu.VMEM((NCPAD, CH), jnp.int32),
        pltpu.VMEM((CH,), jnp.float32),
        pltpu.VMEM((RPS,), jnp.float32),
        pltpu.VMEM_SHARED((R,), jnp.float32),
    ],
)
def _sc_degree(dst_hbm, out_hbm, didx, ones, stripe, acc):
    c = lax.axis_index("c")
    s = lax.axis_index("s")
    wid = _worker(c, s)
    pltpu.sync_copy(dst_hbm.at[wid], didx)

    def fill(i, carry):
        ones[pl.ds(i * 16, 16)] = jnp.full((16,), 1.0, jnp.float32)
        return carry
    lax.fori_loop(0, CH // 16, fill, 0)

    def zero(i, carry):
        stripe[pl.ds(i * 16, 16)] = jnp.zeros((16,), jnp.float32)
        return carry
    lax.fori_loop(0, RPS // 16, zero, 0)
    pltpu.sync_copy(stripe, acc.at[pl.ds(s * RPS, RPS)])
    plsc.subcore_barrier()

    def body(j, carry):
        pltpu.sync_copy(ones, acc.at[didx.at[j]], add=True)
        return carry
    lax.fori_loop(0, NCHUNK, body, 0)
    plsc.subcore_barrier()

    pltpu.sync_copy(acc.at[pl.ds(s * RPS, RPS)], stripe)
    pltpu.sync_copy(stripe, out_hbm.at[c, pl.ds(s * RPS, RPS)])


@functools.partial(
    pl.kernel,
    out_type=jax.ShapeDtypeStruct((NC, R, D), jnp.float32),
    mesh=plsc.VectorSubcoreMesh(**_MESH),
    scratch_types=[
        pltpu.VMEM((NCPAD, CH), jnp.int32),
        pltpu.VMEM((2, CH), jnp.int32),
        pltpu.VMEM((2, CH, D), jnp.float32),
        pltpu.VMEM_SHARED((R, D), jnp.float32),
        pltpu.SemaphoreType.DMA,
        pltpu.SemaphoreType.DMA,
        pltpu.SemaphoreType.DMA,
        pltpu.SemaphoreType.DMA,
    ],
)
def _sc_aggregate(y_hbm, src_hbm, dst_hbm, out_hbm, sidx, didx, rows,
                  acc, rsem0, rsem1, dsem0, dsem1):
    rsems = (rsem0, rsem1)
    dsems = (dsem0, dsem1)
    c = lax.axis_index("c")
    s = lax.axis_index("s")
    wid = _worker(c, s)
    pltpu.async_copy(src_hbm.at[wid], sidx, rsem0)

    def zrow(r, carry):
        for j in range(LANES):
            rows[0, r, pl.ds(j * 16, 16)] = jnp.zeros((16,), jnp.float32)
        return carry
    lax.fori_loop(0, CH, zrow, 0)

    def zacc(k, carry):
        pltpu.sync_copy(rows.at[0],
                        acc.at[pl.ds((s * (RPS // CH) + k) * CH, CH)])
        return carry
    lax.fori_loop(0, RPS // CH, zacc, 0)
    pltpu.make_async_copy(src_hbm.at[wid], sidx, rsem0).wait()
    plsc.subcore_barrier()

    def _issue(j, p):
        pltpu.async_copy(y_hbm.at[sidx.at[j]], rows.at[p], rsems[p])
        pltpu.async_copy(dst_hbm.at[wid, j], didx.at[p], dsems[p])

    def _wait(j, p):
        pltpu.make_async_copy(y_hbm.at[sidx.at[j]], rows.at[p],
                              rsems[p]).wait()
        pltpu.make_async_copy(dst_hbm.at[wid, j], didx.at[p],
                              dsems[p]).wait()

    _issue(0, 0)
    _issue(1, 1)

    def pair(g, carry):
        for p in range(2):
            j = g * 2 + p
            _wait(j, p)
            pltpu.sync_copy(rows.at[p], acc.at[didx.at[p]], add=True)
            _issue(j + 2, p)
        return carry
    lax.fori_loop(0, NCHUNK // 2, pair, 0)
    for p in range(2):
        _wait(NCHUNK + p, p)
    plsc.subcore_barrier()

    for k in range(RPS // CH):
        p = k % 2
        off = (s * (RPS // CH) + k) * CH
        if k >= 2:
            poff = (s * (RPS // CH) + k - 2) * CH
            pltpu.make_async_copy(rows.at[p], out_hbm.at[c, pl.ds(poff, CH)],
                                  rsems[p]).wait()
        pltpu.sync_copy(acc.at[pl.ds(off, CH)], rows.at[p])
        pltpu.async_copy(rows.at[p], out_hbm.at[c, pl.ds(off, CH)], rsems[p])
    for k in range(RPS // CH - 2, RPS // CH):
        p = k % 2
        off = (s * (RPS // CH) + k) * CH
        pltpu.make_async_copy(rows.at[p], out_hbm.at[c, pl.ds(off, CH)],
                              rsems[p]).wait()


def _row_spec():
    return pl.BlockSpec((BM, D), lambda i: (i, 0))


def _vec_spec():
    return pl.BlockSpec((BM,), lambda i: (i,))


def _full_spec(shape):
    return pl.BlockSpec(shape, lambda i: tuple(0 for _ in shape))


def _tc_first_body(x_ref, w_ref, d0_ref, d1_ref, dinv_ref, y_ref):
    dv = lax.rsqrt(d0_ref[...] + d1_ref[...] + 1.0)
    dinv_ref[...] = dv
    y_ref[...] = dv[:, None] * jnp.dot(x_ref[...], w_ref[...],
                                       preferred_element_type=jnp.float32)


def _tc_first(xp, w1, deg0, deg1):
    return pl.pallas_call(
        _tc_first_body,
        grid=(GRID,),
        in_specs=[_row_spec(), _full_spec((D, D)), _vec_spec(), _vec_spec()],
        out_specs=[_vec_spec(), _row_spec()],
        out_shape=[jax.ShapeDtypeStruct((R,), jnp.float32),
                   jax.ShapeDtypeStruct((R, D), jnp.float32)],
    )(xp, w1, deg0, deg1)


def _tc_mid_body(a0_ref, a1_ref, y_ref, dv_ref, b_ref, g_ref, bt_ref, w_ref,
                 out_ref):
    dv = dv_ref[...]
    z = dv[:, None] * (a0_ref[...] + a1_ref[...] + y_ref[...]) + b_ref[...]
    h = jnp.maximum(z * SBN * g_ref[...] + bt_ref[...], 0.0)
    out_ref[...] = dv[:, None] * jnp.dot(h, w_ref[...],
                                         preferred_element_type=jnp.float32)


def _tc_mid(a0, a1, y, dinv, b, g, bt, w_next):
    return pl.pallas_call(
        _tc_mid_body,
        grid=(GRID,),
        in_specs=[_row_spec(), _row_spec(), _row_spec(), _vec_spec(),
                  _full_spec((D,)), _full_spec((D,)), _full_spec((D,)),
                  _full_spec((D, D))],
        out_specs=_row_spec(),
        out_shape=jax.ShapeDtypeStruct((R, D), jnp.float32),
    )(a0, a1, y, dinv, b, g, bt, w_next)


def _tc_final_body(a0_ref, a1_ref, y_ref, dv_ref, b_ref, out_ref):
    dv = dv_ref[...]
    z = dv[:, None] * (a0_ref[...] + a1_ref[...] + y_ref[...]) + b_ref[...]
    z = jnp.clip(z, -4.0, 4.0)
    out_ref[...] = 1.0 / (1.0 + jnp.exp(-z))


def _tc_final(a0, a1, y, dinv, b):
    return pl.pallas_call(
        _tc_final_body,
        grid=(GRID,),
        in_specs=[_row_spec(), _row_spec(), _row_spec(), _vec_spec(),
                  _full_spec((D,))],
        out_specs=_row_spec(),
        out_shape=jax.ShapeDtypeStruct((R, D), jnp.float32),
    )(a0, a1, y, dinv, b)


def kernel(x, edge_index, W1, b1, W2, b2, W3, b3, g1, bt1, g2, bt2):
    src, dst = edge_index[0], edge_index[1]
    npad = EPT_PAD - EPT
    pad_src = jnp.broadcast_to((jnp.arange(npad, dtype=jnp.int32) * 37) % N,
                               (NW, npad))
    pad_dst = jnp.broadcast_to(N + jnp.arange(npad, dtype=jnp.int32) % (R - N),
                               (NW, npad))
    src_p = jnp.concatenate([src.reshape(NW, EPT), pad_src], axis=1)
    dst_p = jnp.concatenate([dst.reshape(NW, EPT), pad_dst], axis=1)
    src_t = src_p.reshape(NW, NCPAD, CH)
    dst_t = dst_p.reshape(NW, NCPAD, CH)
    xp = jnp.pad(x, ((0, R - N), (0, 0)))

    deg = _sc_degree(dst_t)
    dinv, y1 = _tc_first(xp, W1, deg[0], deg[1])
    a1 = _sc_aggregate(y1, src_t, dst_t)
    y2 = _tc_mid(a1[0], a1[1], y1, dinv, b1, g1, bt1, W2)
    a2 = _sc_aggregate(y2, src_t, dst_t)
    y3 = _tc_mid(a2[0], a2[1], y2, dinv, b2, g2, bt2, W3)
    a3 = _sc_aggregate(y3, src_t, dst_t)
    out = _tc_final(a3[0], a3[1], y3, dinv, b3)
    return out[:N]


# speedup vs baseline: 1.0359x; 1.0111x over previous
import functools

import jax
import jax.numpy as jnp
from jax import lax
from jax.experimental import pallas as pl
from jax.experimental.pallas import tpu as pltpu
from jax.experimental.pallas import tpu_sc as plsc

N = 10000
D = 128
E = 320000
NC = 2
NS = 16
NW = NC * NS
R = 10240
CH = 128
EPT = E // NW
NCHUNK = 80
NCPAD = NCHUNK + 2
EPT_PAD = NCPAD * CH
RPS = R // NS
LANES = D // 16
BM = 1024
GRID = R // BM
SBN = 0.9999950000374997

_MESH = dict(core_axis_name="c", subcore_axis_name="s", num_cores=NC,
             num_subcores=NS)


def _worker(c, s):
    return c * NS + s


@functools.partial(
    pl.kernel,
    out_type=jax.ShapeDtypeStruct((NC, R), jnp.float32),
    mesh=plsc.VectorSubcoreMesh(**_MESH),
    scratch_types=[
        pltpu.VMEM((NCPAD, CH), jnp.int32),
        pltpu.VMEM((CH,), jnp.float32),
        pltpu.VMEM((RPS,), jnp.float32),
        pltpu.VMEM_SHARED((R,), jnp.float32),
        pltpu.SemaphoreType.DMA,
        pltpu.SemaphoreType.DMA,
    ],
)
def _sc_degree(dst_hbm, out_hbm, didx, ones, stripe, acc, sem0, sem1):
    c = lax.axis_index("c")
    s = lax.axis_index("s")
    wid = _worker(c, s)
    pltpu.sync_copy(dst_hbm.at[wid], didx)

    def fill(i, carry):
        ones[pl.ds(i * 16, 16)] = jnp.full((16,), 1.0, jnp.float32)
        return carry
    lax.fori_loop(0, CH // 16, fill, 0)

    def zero(i, carry):
        stripe[pl.ds(i * 16, 16)] = jnp.zeros((16,), jnp.float32)
        return carry
    lax.fori_loop(0, RPS // 16, zero, 0)
    pltpu.sync_copy(stripe, acc.at[pl.ds(s * RPS, RPS)])
    plsc.subcore_barrier()

    sems = (sem0, sem1)
    pltpu.async_copy(ones, acc.at[didx.at[0]], sems[0], add=True)
    pltpu.async_copy(ones, acc.at[didx.at[1]], sems[1], add=True)

    def body(g, carry):
        for p in range(2):
            j = g * 2 + p
            pltpu.make_async_copy(ones, acc.at[didx.at[j]], sems[p]).wait()
            pltpu.async_copy(ones, acc.at[didx.at[j + 2]], sems[p], add=True)
        return carry
    lax.fori_loop(0, NCHUNK // 2 - 1, body, 0)
    for p in range(2):
        pltpu.make_async_copy(ones, acc.at[didx.at[NCHUNK - 2 + p]],
                              sems[p]).wait()
    plsc.subcore_barrier()

    pltpu.sync_copy(acc.at[pl.ds(s * RPS, RPS)], stripe)
    pltpu.sync_copy(stripe, out_hbm.at[c, pl.ds(s * RPS, RPS)])


@functools.partial(
    pl.kernel,
    out_type=jax.ShapeDtypeStruct((NC, R, D), jnp.float32),
    mesh=plsc.VectorSubcoreMesh(**_MESH),
    scratch_types=[
        pltpu.VMEM((NCPAD, CH), jnp.int32),
        pltpu.VMEM((2, CH), jnp.int32),
        pltpu.VMEM((2, CH, D), jnp.float32),
        pltpu.VMEM_SHARED((R, D), jnp.float32),
        pltpu.SemaphoreType.DMA,
        pltpu.SemaphoreType.DMA,
        pltpu.SemaphoreType.DMA,
        pltpu.SemaphoreType.DMA,
    ],
)
def _sc_aggregate(y_hbm, src_hbm, dst_hbm, out_hbm, sidx, didx, rows,
                  acc, rsem0, rsem1, dsem0, dsem1):
    rsems = (rsem0, rsem1)
    dsems = (dsem0, dsem1)
    c = lax.axis_index("c")
    s = lax.axis_index("s")
    wid = _worker(c, s)
    pltpu.async_copy(src_hbm.at[wid], sidx, rsem0)

    def zrow(r, carry):
        for j in range(LANES):
            rows[0, r, pl.ds(j * 16, 16)] = jnp.zeros((16,), jnp.float32)
        return carry
    lax.fori_loop(0, CH, zrow, 0)

    def zacc(k, carry):
        pltpu.sync_copy(rows.at[0],
                        acc.at[pl.ds((s * (RPS // CH) + k) * CH, CH)])
        return carry
    lax.fori_loop(0, RPS // CH, zacc, 0)
    pltpu.make_async_copy(src_hbm.at[wid], sidx, rsem0).wait()
    plsc.subcore_barrier()

    def _issue(j, p):
        pltpu.async_copy(y_hbm.at[sidx.at[j]], rows.at[p], rsems[p])
        pltpu.async_copy(dst_hbm.at[wid, j], didx.at[p], dsems[p])

    def _wait(j, p):
        pltpu.make_async_copy(y_hbm.at[sidx.at[j]], rows.at[p],
                              rsems[p]).wait()
        pltpu.make_async_copy(dst_hbm.at[wid, j], didx.at[p],
                              dsems[p]).wait()

    _issue(0, 0)
    _issue(1, 1)

    def pair(g, carry):
        for p in range(2):
            j = g * 2 + p
            _wait(j, p)
            pltpu.sync_copy(rows.at[p], acc.at[didx.at[p]], add=True)
            _issue(j + 2, p)
        return carry
    lax.fori_loop(0, NCHUNK // 2, pair, 0)
    for p in range(2):
        _wait(NCHUNK + p, p)
    plsc.subcore_barrier()

    for k in range(RPS // CH):
        p = k % 2
        off = (s * (RPS // CH) + k) * CH
        if k >= 2:
            poff = (s * (RPS // CH) + k - 2) * CH
            pltpu.make_async_copy(rows.at[p], out_hbm.at[c, pl.ds(poff, CH)],
                                  rsems[p]).wait()
        pltpu.sync_copy(acc.at[pl.ds(off, CH)], rows.at[p])
        pltpu.async_copy(rows.at[p], out_hbm.at[c, pl.ds(off, CH)], rsems[p])
    for k in range(RPS // CH - 2, RPS // CH):
        p = k % 2
        off = (s * (RPS // CH) + k) * CH
        pltpu.make_async_copy(rows.at[p], out_hbm.at[c, pl.ds(off, CH)],
                              rsems[p]).wait()


def _row_spec():
    return pl.BlockSpec((BM, D), lambda i: (i, 0))


def _vec_spec():
    return pl.BlockSpec((BM,), lambda i: (i,))


def _full_spec(shape):
    return pl.BlockSpec(shape, lambda i: tuple(0 for _ in shape))


def _tc_first_body(x_ref, w_ref, d0_ref, d1_ref, dinv_ref, y_ref):
    dv = lax.rsqrt(d0_ref[...] + d1_ref[...] + 1.0)
    dinv_ref[...] = dv
    y_ref[...] = dv[:, None] * jnp.dot(x_ref[...], w_ref[...],
                                       preferred_element_type=jnp.float32)


def _tc_first(xp, w1, deg0, deg1):
    return pl.pallas_call(
        _tc_first_body,
        grid=(GRID,),
        in_specs=[_row_spec(), _full_spec((D, D)), _vec_spec(), _vec_spec()],
        out_specs=[_vec_spec(), _row_spec()],
        out_shape=[jax.ShapeDtypeStruct((R,), jnp.float32),
                   jax.ShapeDtypeStruct((R, D), jnp.float32)],
    )(xp, w1, deg0, deg1)


def _tc_mid_body(a0_ref, a1_ref, y_ref, dv_ref, b_ref, g_ref, bt_ref, w_ref,
                 out_ref):
    dv = dv_ref[...]
    z = dv[:, None] * (a0_ref[...] + a1_ref[...] + y_ref[...]) + b_ref[...]
    h = jnp.maximum(z * SBN * g_ref[...] + bt_ref[...], 0.0)
    out_ref[...] = dv[:, None] * jnp.dot(h, w_ref[...],
                                         preferred_element_type=jnp.float32)


def _tc_mid(a0, a1, y, dinv, b, g, bt, w_next):
    return pl.pallas_call(
        _tc_mid_body,
        grid=(GRID,),
        in_specs=[_row_spec(), _row_spec(), _row_spec(), _vec_spec(),
                  _full_spec((D,)), _full_spec((D,)), _full_spec((D,)),
                  _full_spec((D, D))],
        out_specs=_row_spec(),
        out_shape=jax.ShapeDtypeStruct((R, D), jnp.float32),
    )(a0, a1, y, dinv, b, g, bt, w_next)


def _tc_final_body(a0_ref, a1_ref, y_ref, dv_ref, b_ref, out_ref):
    dv = dv_ref[...]
    z = dv[:, None] * (a0_ref[...] + a1_ref[...] + y_ref[...]) + b_ref[...]
    z = jnp.clip(z, -4.0, 4.0)
    out_ref[...] = 1.0 / (1.0 + jnp.exp(-z))


def _tc_final(a0, a1, y, dinv, b):
    return pl.pallas_call(
        _tc_final_body,
        grid=(GRID,),
        in_specs=[_row_spec(), _row_spec(), _row_spec(), _vec_spec(),
                  _full_spec((D,))],
        out_specs=_row_spec(),
        out_shape=jax.ShapeDtypeStruct((R, D), jnp.float32),
    )(a0, a1, y, dinv, b)


def kernel(x, edge_index, W1, b1, W2, b2, W3, b3, g1, bt1, g2, bt2):
    src, dst = edge_index[0], edge_index[1]
    npad = EPT_PAD - EPT
    pad_src = jnp.broadcast_to((jnp.arange(npad, dtype=jnp.int32) * 37) % N,
                               (NW, npad))
    pad_dst = jnp.broadcast_to(N + jnp.arange(npad, dtype=jnp.int32) % (R - N),
                               (NW, npad))
    src_p = jnp.concatenate([src.reshape(NW, EPT), pad_src], axis=1)
    dst_p = jnp.concatenate([dst.reshape(NW, EPT), pad_dst], axis=1)
    src_t = src_p.reshape(NW, NCPAD, CH)
    dst_t = dst_p.reshape(NW, NCPAD, CH)
    xp = jnp.pad(x, ((0, R - N), (0, 0)))

    deg = _sc_degree(dst_t)
    dinv, y1 = _tc_first(xp, W1, deg[0], deg[1])
    a1 = _sc_aggregate(y1, src_t, dst_t)
    y2 = _tc_mid(a1[0], a1[1], y1, dinv, b1, g1, bt1, W2)
    a2 = _sc_aggregate(y2, src_t, dst_t)
    y3 = _tc_mid(a2[0], a2[1], y2, dinv, b2, g2, bt2, W3)
    a3 = _sc_aggregate(y3, src_t, dst_t)
    out = _tc_final(a3[0], a3[1], y3, dinv, b3)
    return out[:N]
